# baseline (speedup 1.0000x reference)
import functools

import jax
import jax.numpy as jnp
from jax import lax
from jax.experimental import pallas as pl
from jax.experimental.pallas import tpu as pltpu
from jax.experimental.pallas import tpu_sc as plsc

E_EDGES = 160000
C_FEAT = 128
NB = 4

NUM_CORES = 2
NUM_SUBCORES = 16
NUM_WORKERS = NUM_CORES * NUM_SUBCORES
CHUNK = 80


C_PACK = C_FEAT // 2


def _sc_gather(x, idx):
  total = idx.shape[0]
  per_w = total // NUM_WORKERS
  n_chunks = per_w // CHUNK
  assert per_w * NUM_WORKERS == total and n_chunks * CHUNK == per_w
  mesh = plsc.VectorSubcoreMesh(
      core_axis_name="c", subcore_axis_name="s",
      num_cores=NUM_CORES, num_subcores=NUM_SUBCORES)

  @functools.partial(
      pl.kernel,
      mesh=mesh,
      out_type=jax.ShapeDtypeStruct((total, C_PACK), jnp.int32),
      scratch_types=[
          pltpu.VMEM((2, CHUNK), jnp.int32),
          pltpu.VMEM((2, CHUNK, C_PACK), jnp.int32),
          pltpu.SemaphoreType.DMA,
          pltpu.SemaphoreType.DMA,
      ],
      compiler_params=pltpu.CompilerParams(use_tc_tiling_on_sc=False),
  )
  def k(x_hbm, idx_hbm, out_hbm, idx_v, rows_v, gsem0, gsem1):
    wid = lax.axis_index("s") * NUM_CORES + lax.axis_index("c")
    base = pl.multiple_of(wid * per_w, CHUNK)
    gsems = (gsem0, gsem1)
    emax = jnp.full((16,), E_EDGES - 1, jnp.int32)
    ezero = jnp.zeros((16,), jnp.int32)

    def load_idx_and_start(c, b):
      start = pl.multiple_of(base + c * CHUNK, CHUNK)
      pltpu.sync_copy(idx_hbm.at[pl.ds(start, CHUNK)], idx_v.at[b])
      ib = idx_v.at[b]
      for v in range(CHUNK // 16):
        seg = ib[pl.ds(v * 16, 16)]
        ib[pl.ds(v * 16, 16)] = jnp.minimum(jnp.maximum(seg, ezero), emax)
      pltpu.make_async_copy(x_hbm.at[ib], rows_v.at[b], gsems[b]).start()

    def wait_and_writeback(c, b):
      pltpu.make_async_copy(x_hbm.at[idx_v.at[b]], rows_v.at[b],
                            gsems[b]).wait()
      start = pl.multiple_of(base + c * CHUNK, CHUNK)
      pltpu.sync_copy(rows_v.at[b], out_hbm.at[pl.ds(start, CHUNK)])

    for b in (0, 1):
      load_idx_and_start(b, b)

    def body(j, carry):
      for b in (0, 1):
        c = 2 * j + b
        wait_and_writeback(c, b)
        load_idx_and_start(c + 2, b)
      return carry

    lax.fori_loop(0, n_chunks // 2 - 1, body, 0)
    for b in (0, 1):
      wait_and_writeback(n_chunks - 2 + b, b)

  return k(x, idx)


EB = 640
GRID = E_EDGES // EB


def _pack_body(x_ref, o_ref):
  v = x_ref[...]
  lo = jax.lax.bitcast_convert_type(
      v[:, :C_PACK].astype(jnp.bfloat16), jnp.uint16).astype(jnp.uint32)
  hi = jax.lax.bitcast_convert_type(
      v[:, C_PACK:].astype(jnp.bfloat16), jnp.uint16).astype(jnp.uint32)
  o_ref[...] = jax.lax.bitcast_convert_type((hi << 16) | lo, jnp.int32)


def _tc_pack(x):
  return pl.pallas_call(
      _pack_body,
      grid=(GRID,),
      in_specs=[pl.BlockSpec((EB, C_FEAT), lambda i: (i, 0))],
      out_specs=pl.BlockSpec((EB, C_PACK), lambda i: (i, 0)),
      out_shape=jax.ShapeDtypeStruct((E_EDGES, C_PACK), jnp.int32),
  )(x)


def _unpack(p):
  u = jax.lax.bitcast_convert_type(p, jnp.uint32)
  lo = jax.lax.bitcast_convert_type(
      (u & jnp.uint32(0xFFFF)).astype(jnp.uint16), jnp.bfloat16)
  hi = jax.lax.bitcast_convert_type(
      (u >> 16).astype(jnp.uint16), jnp.bfloat16)
  return jnp.concatenate([lo, hi], axis=1)


def _mm_body(x_ref, n0_ref, n1_ref, n2_ref, n3_ref, wt_ref, y_ref, st_ref):
  i = pl.program_id(0)
  n0, n1 = _unpack(n0_ref[...]), _unpack(n1_ref[...])
  n2, n3 = _unpack(n2_ref[...]), _unpack(n3_ref[...])
  feats = (_unpack(x_ref[...]),
           jnp.minimum(n0, n1), jnp.maximum(n0, n1),
           jnp.minimum(n2, n3), jnp.maximum(n2, n3))
  y = jnp.zeros((EB, C_FEAT), jnp.float32)
  for j, f in enumerate(feats):
    y = y + jnp.dot(f, wt_ref[j * C_FEAT:(j + 1) * C_FEAT, :],
                    preferred_element_type=jnp.float32)
  y_ref[...] = y.astype(jnp.bfloat16)

  @pl.when(i == 0)
  def _():
    st_ref[...] = jnp.zeros_like(st_ref)

  st_ref[0:1, :] += jnp.sum(y, axis=0, keepdims=True)
  st_ref[1:2, :] += jnp.sum(y * y, axis=0, keepdims=True)


def _tc_matmul_stats(x, g, wt):
  def plane_spec(j):
    return pl.BlockSpec((EB, C_PACK), lambda i, j=j: (j * GRID + i, 0))

  return pl.pallas_call(
      _mm_body,
      grid=(GRID,),
      in_specs=[
          pl.BlockSpec((EB, C_PACK), lambda i: (i, 0)),
          plane_spec(0), plane_spec(1), plane_spec(2), plane_spec(3),
          pl.BlockSpec((5 * C_FEAT, C_FEAT), lambda i: (0, 0)),
      ],
      out_specs=[
          pl.BlockSpec((EB, C_FEAT), lambda i: (i, 0)),
          pl.BlockSpec((8, C_FEAT), lambda i: (0, 0)),
      ],
      out_shape=[
          jax.ShapeDtypeStruct((E_EDGES, C_FEAT), jnp.bfloat16),
          jax.ShapeDtypeStruct((8, C_FEAT), jnp.float32),
      ],
  )(x, g, g, g, g, wt)


def _bn_body(y_ref, st_ref, gb_ref, o_ref):
  inv_e = jnp.float32(1.0 / E_EDGES)
  mean = st_ref[0, :] * inv_e
  var = st_ref[1, :] * inv_e - mean * mean
  inv = lax.rsqrt(var + 1e-5)
  scale = gb_ref[0, :] * inv
  shift = gb_ref[1, :] - mean * scale
  yv = y_ref[...].astype(jnp.float32)
  o_ref[...] = jnp.maximum(yv * scale[None, :] + shift[None, :], 0.0)


def _tc_bn_relu(y, st, gb):
  return pl.pallas_call(
      _bn_body,
      grid=(GRID,),
      in_specs=[
          pl.BlockSpec((EB, C_FEAT), lambda i: (i, 0)),
          pl.BlockSpec((8, C_FEAT), lambda i: (0, 0)),
          pl.BlockSpec((8, C_FEAT), lambda i: (0, 0)),
      ],
      out_specs=pl.BlockSpec((EB, C_FEAT), lambda i: (i, 0)),
      out_shape=jax.ShapeDtypeStruct((E_EDGES, C_FEAT), jnp.float32),
  )(y, st, gb)


def kernel(x, nb, W, gamma, beta):
  idx = nb.astype(jnp.int32).T.reshape(-1)
  xp = _tc_pack(x)
  g = _sc_gather(xp, idx)
  wt = W.T.astype(jnp.bfloat16)
  y, st = _tc_matmul_stats(xp, g, wt)
  gb = jnp.zeros((8, C_FEAT), jnp.float32).at[0].set(gamma).at[1].set(beta)
  return _tc_bn_relu(y, st, gb)


# speedup vs baseline: 1.8532x; 1.8532x over previous
import functools

import jax
import jax.numpy as jnp
from jax import lax
from jax.experimental import pallas as pl
from jax.experimental.pallas import tpu as pltpu
from jax.experimental.pallas import tpu_sc as plsc

E_EDGES = 160000
C_FEAT = 128
NB = 4

NUM_CORES = 2
NUM_SUBCORES = 16
NUM_WORKERS = NUM_CORES * NUM_SUBCORES
CHUNK = 80

N_SLICES = 5
E_SLICE = E_EDGES // N_SLICES

EB = 640
GRID = E_EDGES // EB
SBLK = E_SLICE // EB


def _sc_gather(x, idx):
  total = idx.shape[0]
  per_w = total // NUM_WORKERS
  n_chunks = per_w // CHUNK
  assert per_w * NUM_WORKERS == total and n_chunks * CHUNK == per_w
  mesh = plsc.VectorSubcoreMesh(
      core_axis_name="c", subcore_axis_name="s",
      num_cores=NUM_CORES, num_subcores=NUM_SUBCORES)

  @functools.partial(
      pl.kernel,
      mesh=mesh,
      out_type=jax.ShapeDtypeStruct((total, C_FEAT), jnp.float32),
      scratch_types=[
          pltpu.VMEM((2, CHUNK), jnp.int32),
          pltpu.VMEM((2, CHUNK, C_FEAT), jnp.float32),
          pltpu.SemaphoreType.DMA,
          pltpu.SemaphoreType.DMA,
      ],
  )
  def k(x_hbm, idx_hbm, out_hbm, idx_v, rows_v, gsem0, gsem1):
    wid = lax.axis_index("s") * NUM_CORES + lax.axis_index("c")
    base = pl.multiple_of(wid * per_w, CHUNK)
    gsems = (gsem0, gsem1)
    emax = jnp.full((16,), E_EDGES - 1, jnp.int32)
    ezero = jnp.zeros((16,), jnp.int32)

    def load_idx_and_start(c, b):
      start = pl.multiple_of(base + c * CHUNK, CHUNK)
      pltpu.sync_copy(idx_hbm.at[pl.ds(start, CHUNK)], idx_v.at[b])
      ib = idx_v.at[b]
      for v in range(CHUNK // 16):
        seg = ib[pl.ds(v * 16, 16)]
        ib[pl.ds(v * 16, 16)] = jnp.minimum(jnp.maximum(seg, ezero), emax)
      pltpu.make_async_copy(x_hbm.at[ib], rows_v.at[b], gsems[b]).start()

    def wait_and_writeback(c, b):
      pltpu.make_async_copy(x_hbm.at[idx_v.at[b]], rows_v.at[b],
                            gsems[b]).wait()
      start = pl.multiple_of(base + c * CHUNK, CHUNK)
      pltpu.sync_copy(rows_v.at[b], out_hbm.at[pl.ds(start, CHUNK)])

    for b in (0, 1):
      load_idx_and_start(b, b)

    def body(j, carry):
      for b in (0, 1):
        c = 2 * j + b
        wait_and_writeback(c, b)
        load_idx_and_start(c + 2, b)
      return carry

    lax.fori_loop(0, n_chunks // 2 - 1, body, 0)
    for b in (0, 1):
      wait_and_writeback(n_chunks - 2 + b, b)

  return k(x, idx)


def _mm_body(x_ref, n0_ref, n1_ref, n2_ref, n3_ref, wt_ref, *rest):
  if len(rest) == 3:
    _, y_ref, st_ref = rest
  else:
    y_ref, st_ref = rest
  i = pl.program_id(0)
  n0, n1, n2, n3 = n0_ref[...], n1_ref[...], n2_ref[...], n3_ref[...]
  feats = (x_ref[...],
           jnp.minimum(n0, n1), jnp.maximum(n0, n1),
           jnp.minimum(n2, n3), jnp.maximum(n2, n3))
  y = jnp.zeros((EB, C_FEAT), jnp.float32)
  for j, f in enumerate(feats):
    y = y + jnp.dot(f, wt_ref[j * C_FEAT:(j + 1) * C_FEAT, :],
                    preferred_element_type=jnp.float32)
  y_ref[...] = y.astype(jnp.bfloat16)

  @pl.when(i == 0)
  def _():
    st_ref[...] = jnp.zeros_like(st_ref)

  st_ref[0:1, :] += jnp.sum(y, axis=0, keepdims=True)
  st_ref[1:2, :] += jnp.sum(y * y, axis=0, keepdims=True)


def _tc_matmul_stats_slice(x, g, wt, y_prev, s):
  def plane_spec(j):
    return pl.BlockSpec((EB, C_FEAT), lambda i, j=j: (j * SBLK + i, 0))

  in_specs = [
      pl.BlockSpec((EB, C_FEAT), lambda i: (s * SBLK + i, 0)),
      plane_spec(0), plane_spec(1), plane_spec(2), plane_spec(3),
      pl.BlockSpec((5 * C_FEAT, C_FEAT), lambda i: (0, 0)),
  ]
  args = [x, g, g, g, g, wt]
  kwargs = {}
  if y_prev is not None:
    in_specs.append(pl.BlockSpec((8, C_FEAT), lambda i: (0, 0)))
    args.append(y_prev)
    kwargs["input_output_aliases"] = {6: 0}
  return pl.pallas_call(
      _mm_body,
      grid=(SBLK,),
      in_specs=in_specs,
      out_specs=[
          pl.BlockSpec((EB, C_FEAT), lambda i: (s * SBLK + i, 0)),
          pl.BlockSpec((8, C_FEAT), lambda i: (0, 0)),
      ],
      out_shape=[
          jax.ShapeDtypeStruct((E_EDGES, C_FEAT), jnp.bfloat16),
          jax.ShapeDtypeStruct((8, C_FEAT), jnp.float32),
      ],
      **kwargs,
  )(*args)


def _bn_body(y_ref, st0, st1, st2, st3, st4, gb_ref, o_ref):
  st = st0[...] + st1[...] + st2[...] + st3[...] + st4[...]
  inv_e = jnp.float32(1.0 / E_EDGES)
  mean = st[0, :] * inv_e
  var = st[1, :] * inv_e - mean * mean
  inv = lax.rsqrt(var + 1e-5)
  scale = gb_ref[0, :] * inv
  shift = gb_ref[1, :] - mean * scale
  yv = y_ref[...].astype(jnp.float32)
  o_ref[...] = jnp.maximum(yv * scale[None, :] + shift[None, :], 0.0)


def _tc_bn_relu(y, sts, gb):
  small = pl.BlockSpec((8, C_FEAT), lambda i: (0, 0))
  return pl.pallas_call(
      _bn_body,
      grid=(GRID,),
      in_specs=[pl.BlockSpec((EB, C_FEAT), lambda i: (i, 0))]
      + [small] * (len(sts) + 1),
      out_specs=pl.BlockSpec((EB, C_FEAT), lambda i: (i, 0)),
      out_shape=jax.ShapeDtypeStruct((E_EDGES, C_FEAT), jnp.float32),
  )(y, *sts, gb)


def kernel(x, nb, W, gamma, beta):
  idx = (nb.astype(jnp.int32).T
         .reshape(NB, N_SLICES, E_SLICE)
         .transpose(1, 0, 2)
         .reshape(N_SLICES, NB * E_SLICE))
  wt = W.T
  y = None
  sts = []
  for s in range(N_SLICES):
    g = _sc_gather(x, idx[s])
    y, st = _tc_matmul_stats_slice(x, g, wt, y, s)
    sts.append(st)
  gb = jnp.zeros((8, C_FEAT), jnp.float32).at[0].set(gamma).at[1].set(beta)
  return _tc_bn_relu(y, sts, gb)
